# baseline (speedup 1.0000x reference)
import jax
import jax.numpy as jnp
from jax import lax
from jax.experimental import pallas as pl
from jax.experimental.pallas import tpu as pltpu
from jax.experimental.pallas import tpu_sc as plsc

EMB = 32
BATCH = 16384
FIELDS = 26
B_TOTAL = BATCH * FIELDS
VOCAB_N = 1000000
NC = 2
NS = 16
NW = NC * NS
B_PER_W = B_TOTAL // NW
CHUNK = 1024
NCHUNK = B_PER_W // CHUNK

COLS = VOCAB_N // 128
TAIL = VOCAB_N - COLS * 128
VOCAB_PAD = (COLS + 1) * 128


def _detile_body(table_t_hbm, det_hbm, sem):
    wid = lax.axis_index("s") * NC + lax.axis_index("c")
    g = wid // 8
    w = wid % 8
    n = COLS // 8 + jnp.where(w < COLS % 8, 1, 0)
    c0 = w * (COLS // 8) + jnp.minimum(w, COLS % 8)

    npiece = 8
    base = COLS // 8 // npiece

    def piece(i, _):
        lo = c0 + i * base
        pltpu.async_copy(
            table_t_hbm.at[pl.ds(g * 8, 8), pl.ds(lo * 128, base * 128)],
            det_hbm.at[g, :, pl.ds(lo * 128, base * 128)],
            sem,
        )
        return _

    lax.fori_loop(0, npiece, piece, 0)
    rem = n - base * npiece

    @pl.when(rem > 0)
    def _remainder():
        lo = c0 + base * npiece
        pltpu.async_copy(
            table_t_hbm.at[pl.ds(g * 8, 8), pl.ds(lo * 128, 128)],
            det_hbm.at[g, :, pl.ds(lo * 128, 128)],
            sem,
        )

    def drain(i, _):
        pltpu.make_async_copy(
            det_hbm.at[g, :, pl.ds(0, base * 128)],
            det_hbm.at[g, :, pl.ds(0, base * 128)],
            sem,
        ).wait()
        return _

    lax.fori_loop(0, npiece, drain, 0)

    @pl.when(rem > 0)
    def _drain_rem():
        pltpu.make_async_copy(
            det_hbm.at[g, :, pl.ds(0, 128)],
            det_hbm.at[g, :, pl.ds(0, 128)],
            sem,
        ).wait()


def _transpose_body(det_hbm, tail_hbm, scratch_hbm, tile_v, out_v, isem, osem):
    wid = lax.axis_index("s") * NC + lax.axis_index("c")
    lane = lax.iota(jnp.int32, 16)
    base_idx = lane * EMB

    n = COLS // NW + jnp.where(wid < COLS % NW, 1, 0)
    c0 = wid * (COLS // NW) + jnp.minimum(wid, COLS % NW)

    def do_block(col, nkeys):
        copies = [
            pltpu.make_async_copy(
                det_hbm.at[g, :, pl.ds(col * 128, nkeys)],
                tile_v.at[g, :, pl.ds(0, nkeys)],
                isem,
            )
            for g in range(4)
        ]
        for c in copies:
            c.start()
        for c in copies:
            c.wait()

        for g in range(4):
            for r in range(8):
                e = g * 8 + r

                def j_step(jg, __, g=g, r=r, e=e):
                    v = tile_v[g, r, pl.ds(jg * 16, 16)]
                    plsc.store_scatter(out_v, [jg * 512 + base_idx + e], v)
                    return __

                lax.fori_loop(0, nkeys // 16, j_step, 0)

        pltpu.async_copy(
            out_v.at[pl.ds(0, nkeys * EMB)],
            scratch_hbm.at[pl.ds(col * 128 * EMB, nkeys * EMB)],
            osem,
        ).wait()

    BLK = 4

    def block_step(i, _):
        do_block(c0 + i * BLK, BLK * 128)
        return _

    nb = n // BLK
    lax.fori_loop(0, nb, block_step, 0)

    def tail_step(i, _):
        do_block(c0 + nb * BLK + i, 128)
        return _

    lax.fori_loop(0, n - nb * BLK, tail_step, 0)

    @pl.when(wid == 0)
    def _last():
        pltpu.sync_copy(tail_hbm, scratch_hbm.at[pl.ds(COLS * 128 * EMB, TAIL * EMB)])


def _gather_body(table_hbm, idx_hbm, out_hbm, idx_v, rows, gsem, ssem):
    wid = lax.axis_index("s") * NC + lax.axis_index("c")
    base = wid * B_PER_W
    pltpu.sync_copy(idx_hbm.at[pl.ds(base, B_PER_W)], idx_v)

    def gather(i, p):
        return pltpu.make_async_copy(
            table_hbm.at[idx_v.at[pl.ds(i * CHUNK, CHUNK)]], rows.at[p], gsem[p]
        )

    def store(i, p):
        return pltpu.make_async_copy(
            rows.at[p], out_hbm.at[pl.ds(base + i * CHUNK, CHUNK)], ssem[p]
        )

    gather(0, 0).start()
    for i in range(NCHUNK):
        p = i % 2
        if i + 1 < NCHUNK:
            if i >= 1:
                store(i - 1, 1 - p).wait()
            gather(i + 1, 1 - p).start()
        gather(i, p).wait()
        store(i, p).start()
    store(NCHUNK - 2, NCHUNK % 2).wait()
    store(NCHUNK - 1, (NCHUNK - 1) % 2).wait()


@jax.jit
def _lookup(table, idx):
    mesh = plsc.VectorSubcoreMesh(core_axis_name="c", subcore_axis_name="s")
    table_t = jnp.transpose(table)
    tail = lax.slice(table, (COLS * 128, 0), (VOCAB_N, EMB)).reshape(-1)
    detiled = pl.kernel(
        _detile_body,
        out_type=jax.ShapeDtypeStruct((4, 8, VOCAB_PAD), jnp.float32),
        mesh=mesh,
        scratch_types=[pltpu.SemaphoreType.DMA],
        compiler_params=pltpu.CompilerParams(use_tc_tiling_on_sc=True),
    )(table_t)
    scratch = pl.kernel(
        _transpose_body,
        out_type=jax.ShapeDtypeStruct((VOCAB_PAD * EMB,), jnp.float32),
        mesh=mesh,
        scratch_types=[
            pltpu.VMEM((4, 8, 512), jnp.float32),
            pltpu.VMEM((4 * 128 * EMB,), jnp.float32),
            pltpu.SemaphoreType.DMA,
            pltpu.SemaphoreType.DMA,
        ],
        compiler_params=pltpu.CompilerParams(
            use_tc_tiling_on_sc=True, needs_layout_passes=False
        ),
    )(detiled, tail)
    table_rm = jnp.reshape(scratch, (VOCAB_PAD, EMB))
    return pl.kernel(
        _gather_body,
        out_type=jax.ShapeDtypeStruct((B_TOTAL, EMB), jnp.float32),
        mesh=mesh,
        scratch_types=[
            pltpu.VMEM((B_PER_W,), jnp.int32),
            pltpu.VMEM((2, CHUNK, EMB), jnp.float32),
            (pltpu.SemaphoreType.DMA, pltpu.SemaphoreType.DMA),
            (pltpu.SemaphoreType.DMA, pltpu.SemaphoreType.DMA),
        ],
        compiler_params=pltpu.CompilerParams(use_tc_tiling_on_sc=False),
    )(table_rm, idx)


def kernel(inputs, table):
    idx = inputs.reshape(-1).astype(jnp.int32)
    flat = _lookup(table, idx)
    return flat.reshape(BATCH, FIELDS, EMB)


# speedup vs baseline: 5.0934x; 5.0934x over previous
import jax
import jax.numpy as jnp
from jax import lax
from jax.experimental import pallas as pl
from jax.experimental.pallas import tpu as pltpu
from jax.experimental.pallas import tpu_sc as plsc

EMB = 32
BATCH = 16384
FIELDS = 26
B_TOTAL = BATCH * FIELDS
VOCAB_N = 1000000
NC = 2
NS = 16
NW = NC * NS
B_PER_W = B_TOTAL // NW
CHUNK = 1024
NCHUNK = B_PER_W // CHUNK

COLS = VOCAB_N // 128
TAIL = VOCAB_N - COLS * 128
VOCAB_PAD = (COLS + 1) * 128

BLK = 4
BKEYS = BLK * 128
NB = COLS // NW // BLK
EXTRA = COLS % NW


def _transpose_body(table_t_hbm, tail_hbm, scratch_hbm, tile_v0, tile_v1, out_v0, out_v1, isems, osems):
    wid = lax.axis_index("s") * NC + lax.axis_index("c")
    lane = lax.iota(jnp.int32, 16)
    base_idx = lane * EMB

    c0 = wid * (NB * BLK) + jnp.minimum(wid, EXTRA)

    tiles = (tile_v0, tile_v1)
    outs = (out_v0, out_v1)

    def in_copies(col, b, nkeys):
        return [
            pltpu.make_async_copy(
                table_t_hbm.at[pl.ds(g * 8, 8), pl.ds(col * 128, nkeys)],
                tiles[b].at[g, :, pl.ds(0, nkeys)],
                isems[b],
            )
            for g in range(4)
        ]

    def transpose(b, nkeys):
        for g in range(4):
            for r in range(8):
                e = g * 8 + r
                base_e = base_idx + e

                def jq_step(jq, __, g=g, r=r, base_e=base_e):
                    for u in range(4):
                        jg = jq * 4 + u
                        v = tiles[b][g, r, pl.ds(jg * 16, 16)]
                        plsc.store_scatter(outs[b], [jg * 512 + base_e], v)
                    return __

                lax.fori_loop(0, nkeys // 64, jq_step, 0)

    def out_copy(col, b, nkeys):
        return pltpu.make_async_copy(
            outs[b].at[pl.ds(0, nkeys * EMB)],
            scratch_hbm.at[pl.ds(col * 128 * EMB, nkeys * EMB)],
            osems[b],
        )

    for c in in_copies(c0, 0, BKEYS):
        c.start()

    def block_step(i, carry):
        col = c0 + i * BLK
        for p in range(2):

            @pl.when(lax.rem(i, 2) == p)
            def _run(p=p, col=col):
                for c in in_copies(col, p, BKEYS):
                    c.wait()

                @pl.when(i + 1 < NB)
                def _prefetch():
                    for c in in_copies(col + BLK, 1 - p, BKEYS):
                        c.start()

                @pl.when(i >= 1)
                def _drain_prev():
                    out_copy(col, 1 - p, BKEYS).wait()

                transpose(p, BKEYS)
                out_copy(col, p, BKEYS).start()

        return carry

    lax.fori_loop(0, NB, block_step, 0)
    out_copy(c0, (NB - 1) % 2, BKEYS).wait()

    @pl.when(wid < EXTRA)
    def _extra():
        col = c0 + NB * BLK
        for c in in_copies(col, 0, 128):
            c.start()
        for c in in_copies(col, 0, 128):
            c.wait()
        transpose(0, 128)
        out_copy(col, 0, 128).start()
        out_copy(col, 0, 128).wait()

    @pl.when(wid == 0)
    def _tail():
        pltpu.sync_copy(
            tail_hbm, scratch_hbm.at[pl.ds(COLS * 128 * EMB, TAIL * EMB)]
        )


def _gather_body(table_hbm, idx_hbm, out_hbm, idx_v, rows, gsem, ssem):
    wid = lax.axis_index("s") * NC + lax.axis_index("c")
    base = wid * B_PER_W
    pltpu.sync_copy(idx_hbm.at[pl.ds(base, B_PER_W)], idx_v)

    def gather(i, p):
        return pltpu.make_async_copy(
            table_hbm.at[idx_v.at[pl.ds(i * CHUNK, CHUNK)]], rows.at[p], gsem[p]
        )

    def store(i, p):
        return pltpu.make_async_copy(
            rows.at[p], out_hbm.at[pl.ds(base + i * CHUNK, CHUNK)], ssem[p]
        )

    gather(0, 0).start()
    for i in range(NCHUNK):
        p = i % 2
        if i + 1 < NCHUNK:
            if i >= 1:
                store(i - 1, 1 - p).wait()
            gather(i + 1, 1 - p).start()
        gather(i, p).wait()
        store(i, p).start()
    store(NCHUNK - 2, NCHUNK % 2).wait()
    store(NCHUNK - 1, (NCHUNK - 1) % 2).wait()


@jax.jit
def _lookup(table, idx):
    mesh = plsc.VectorSubcoreMesh(core_axis_name="c", subcore_axis_name="s")
    table_t = jnp.transpose(table)
    tail = lax.slice(table, (COLS * 128, 0), (VOCAB_N, EMB)).reshape(-1)
    scratch = pl.kernel(
        _transpose_body,
        out_type=jax.ShapeDtypeStruct((VOCAB_PAD * EMB,), jnp.float32),
        mesh=mesh,
        scratch_types=[
            pltpu.VMEM((4, 8, BKEYS), jnp.float32),
            pltpu.VMEM((4, 8, BKEYS), jnp.float32),
            pltpu.VMEM((BKEYS * EMB,), jnp.float32),
            pltpu.VMEM((BKEYS * EMB,), jnp.float32),
            (pltpu.SemaphoreType.DMA, pltpu.SemaphoreType.DMA),
            (pltpu.SemaphoreType.DMA, pltpu.SemaphoreType.DMA),
        ],
        compiler_params=pltpu.CompilerParams(
            use_tc_tiling_on_sc=True, needs_layout_passes=False
        ),
    )(table_t, tail)
    table_rm = jnp.reshape(scratch, (VOCAB_PAD, EMB))
    return pl.kernel(
        _gather_body,
        out_type=jax.ShapeDtypeStruct((B_TOTAL, EMB), jnp.float32),
        mesh=mesh,
        scratch_types=[
            pltpu.VMEM((B_PER_W,), jnp.int32),
            pltpu.VMEM((2, CHUNK, EMB), jnp.float32),
            (pltpu.SemaphoreType.DMA, pltpu.SemaphoreType.DMA),
            (pltpu.SemaphoreType.DMA, pltpu.SemaphoreType.DMA),
        ],
        compiler_params=pltpu.CompilerParams(use_tc_tiling_on_sc=False),
    )(table_rm, idx)


def kernel(inputs, table):
    idx = inputs.reshape(-1).astype(jnp.int32)
    flat = _lookup(table, idx)
    return flat.reshape(BATCH, FIELDS, EMB)


# speedup vs baseline: 7.3607x; 1.4451x over previous
import jax
import jax.numpy as jnp
from jax import lax
from jax.experimental import pallas as pl
from jax.experimental.pallas import tpu as pltpu
from jax.experimental.pallas import tpu_sc as plsc

EMB = 32
BATCH = 16384
FIELDS = 26
B_TOTAL = BATCH * FIELDS
VOCAB_N = 1000000
NC = 2
NS = 16
NW = NC * NS
B_PER_W = B_TOTAL // NW
CHUNK = 1024
NCHUNK = B_PER_W // CHUNK

COLS = VOCAB_N // 128
TAIL = VOCAB_N - COLS * 128
VOCAB_PAD = (COLS + 1) * 128

BLK = 4
BKEYS = BLK * 128
NB = COLS // NW // BLK
EXTRA = COLS % NW


def _transpose_body(table_t_hbm, tail_hbm, scratch_hbm, tile_v0, tile_v1, out_v0, out_v1, isems, osems):
    wid = lax.axis_index("s") * NC + lax.axis_index("c")
    lane = lax.iota(jnp.int32, 16)
    base_idx = lane * EMB

    c0 = wid * (NB * BLK) + jnp.minimum(wid, EXTRA)

    tiles = (tile_v0, tile_v1)
    outs = (out_v0, out_v1)

    def in_copies(col, b, nkeys):
        return [
            pltpu.make_async_copy(
                table_t_hbm.at[pl.ds(g * 8, 8), pl.ds(col * 128, nkeys)],
                tiles[b].at[g, :, pl.ds(0, nkeys)],
                isems[b],
            )
            for g in range(4)
        ]

    def transpose(b, nkeys):
        for d in range(16):
            pd = jnp.bitwise_and(lane + d, 15)
            gv0 = lax.shift_right_logical(pd, 3)
            rv = jnp.bitwise_and(pd, 7)
            sv0 = base_idx + pd
            for eg in range(2):
                gv = gv0 + 2 * eg if eg else gv0
                sv = sv0 + 16 * eg if eg else sv0

                def jq_step(jq, __, gv=gv, rv=rv, sv=sv):
                    for u in range(4):
                        jg = jq * 4 + u
                        jv = lane + jg * 16
                        v = plsc.load_gather(tiles[b], [gv, rv, jv])
                        plsc.store_scatter(outs[b], [sv + jg * 512], v)
                    return __

                lax.fori_loop(0, nkeys // 64, jq_step, 0)

    def out_copy(col, b, nkeys):
        return pltpu.make_async_copy(
            outs[b].at[pl.ds(0, nkeys * EMB)],
            scratch_hbm.at[pl.ds(col * 128 * EMB, nkeys * EMB)],
            osems[b],
        )

    for c in in_copies(c0, 0, BKEYS):
        c.start()

    def block_step(i, carry):
        col = c0 + i * BLK
        for p in range(2):

            @pl.when(lax.rem(i, 2) == p)
            def _run(p=p, col=col):
                for c in in_copies(col, p, BKEYS):
                    c.wait()

                @pl.when(i + 1 < NB)
                def _prefetch():
                    for c in in_copies(col + BLK, 1 - p, BKEYS):
                        c.start()

                @pl.when(i >= 1)
                def _drain_prev():
                    out_copy(col, 1 - p, BKEYS).wait()

                transpose(p, BKEYS)
                out_copy(col, p, BKEYS).start()

        return carry

    lax.fori_loop(0, NB, block_step, 0)
    out_copy(c0, (NB - 1) % 2, BKEYS).wait()

    @pl.when(wid < EXTRA)
    def _extra():
        col = c0 + NB * BLK
        for c in in_copies(col, 0, 128):
            c.start()
        for c in in_copies(col, 0, 128):
            c.wait()
        transpose(0, 128)
        out_copy(col, 0, 128).start()
        out_copy(col, 0, 128).wait()

    @pl.when(wid == 0)
    def _tail():
        pltpu.sync_copy(
            tail_hbm, scratch_hbm.at[pl.ds(COLS * 128 * EMB, TAIL * EMB)]
        )


def _gather_body(table_hbm, idx_hbm, out_hbm, idx_v, rows, gsem, ssem):
    wid = lax.axis_index("s") * NC + lax.axis_index("c")
    base = wid * B_PER_W
    pltpu.sync_copy(idx_hbm.at[pl.ds(base, B_PER_W)], idx_v)

    def gather(i, p):
        return pltpu.make_async_copy(
            table_hbm.at[idx_v.at[pl.ds(i * CHUNK, CHUNK)]], rows.at[p], gsem[p]
        )

    def store(i, p):
        return pltpu.make_async_copy(
            rows.at[p], out_hbm.at[pl.ds(base + i * CHUNK, CHUNK)], ssem[p]
        )

    gather(0, 0).start()
    for i in range(NCHUNK):
        p = i % 2
        if i + 1 < NCHUNK:
            if i >= 1:
                store(i - 1, 1 - p).wait()
            gather(i + 1, 1 - p).start()
        gather(i, p).wait()
        store(i, p).start()
    store(NCHUNK - 2, NCHUNK % 2).wait()
    store(NCHUNK - 1, (NCHUNK - 1) % 2).wait()


@jax.jit
def _lookup(table, idx):
    mesh = plsc.VectorSubcoreMesh(core_axis_name="c", subcore_axis_name="s")
    table_t = jnp.transpose(table)
    tail = lax.slice(table, (COLS * 128, 0), (VOCAB_N, EMB)).reshape(-1)
    scratch = pl.kernel(
        _transpose_body,
        out_type=jax.ShapeDtypeStruct((VOCAB_PAD * EMB,), jnp.float32),
        mesh=mesh,
        scratch_types=[
            pltpu.VMEM((4, 8, BKEYS), jnp.float32),
            pltpu.VMEM((4, 8, BKEYS), jnp.float32),
            pltpu.VMEM((BKEYS * EMB,), jnp.float32),
            pltpu.VMEM((BKEYS * EMB,), jnp.float32),
            (pltpu.SemaphoreType.DMA, pltpu.SemaphoreType.DMA),
            (pltpu.SemaphoreType.DMA, pltpu.SemaphoreType.DMA),
        ],
        compiler_params=pltpu.CompilerParams(
            use_tc_tiling_on_sc=True, needs_layout_passes=False
        ),
    )(table_t, tail)
    table_rm = jnp.reshape(scratch, (VOCAB_PAD, EMB))
    return pl.kernel(
        _gather_body,
        out_type=jax.ShapeDtypeStruct((B_TOTAL, EMB), jnp.float32),
        mesh=mesh,
        scratch_types=[
            pltpu.VMEM((B_PER_W,), jnp.int32),
            pltpu.VMEM((2, CHUNK, EMB), jnp.float32),
            (pltpu.SemaphoreType.DMA, pltpu.SemaphoreType.DMA),
            (pltpu.SemaphoreType.DMA, pltpu.SemaphoreType.DMA),
        ],
        compiler_params=pltpu.CompilerParams(use_tc_tiling_on_sc=False),
    )(table_rm, idx)


def kernel(inputs, table):
    idx = inputs.reshape(-1).astype(jnp.int32)
    flat = _lookup(table, idx)
    return flat.reshape(BATCH, FIELDS, EMB)
